# baseline (speedup 1.0000x reference)
import functools

import jax
import jax.numpy as jnp
from jax import lax
from jax.experimental import pallas as pl
from jax.experimental.pallas import tpu as pltpu
from jax.experimental.pallas import tpu_sc as plsc

B = 4096
F = 26
D = 64
ROWS_PER_TABLE = 100000
BF = B * F

NC, NS = 2, 16
NW = NC * NS
PW = BF // NW
CHUNK = 128
NCHUNK = PW // CHUNK


def _emb_gather_body(x_hbm, emb1_hbm, emb2_hbm, w_hbm, v_hbm,
                     x_v, idx_v, r0, r1, w0, w1, s0, s1, s2, s3):
    wid = lax.axis_index("s") * NC + lax.axis_index("c")
    base = wid * PW

    pltpu.sync_copy(x_hbm.at[pl.ds(base, PW)], x_v)

    def idx_body(i, carry):
        p0 = base + i * 16
        lanes = p0 + lax.iota(jnp.int32, 16)
        off = (lanes % F) * ROWS_PER_TABLE
        idx_v[pl.ds(i * 16, 16)] = x_v[pl.ds(i * 16, 16)] + off
        return carry

    lax.fori_loop(0, PW // 16, idx_body, 0)

    bufs = ((r0, w0, s0, s1), (r1, w1, s2, s3))
    descs = [None, None]
    for c in range(NCHUNK + 1):
        if c < NCHUNK:
            rb, wb, sr, sw = bufs[c % 2]
            idx = idx_v.at[pl.ds(c * CHUNK, CHUNK)]
            descs[c % 2] = (
                pltpu.async_copy(emb2_hbm.at[idx], rb, sr),
                pltpu.async_copy(emb1_hbm.at[idx], wb, sw),
            )
        if c >= 1:
            cp = c - 1
            rb, wb, _, _ = bufs[cp % 2]
            d2, d1 = descs[cp % 2]
            d2.wait()
            d1.wait()
            pltpu.sync_copy(rb, v_hbm.at[pl.ds(base + cp * CHUNK, CHUNK)])
            pltpu.sync_copy(wb, w_hbm.at[pl.ds(base + cp * CHUNK, CHUNK)])


def _emb_gather(x_flat, emb1_w, emb2_w):
    mesh = plsc.VectorSubcoreMesh(
        core_axis_name="c", subcore_axis_name="s", num_cores=NC,
        num_subcores=NS)
    return pl.kernel(
        _emb_gather_body,
        out_type=(
            jax.ShapeDtypeStruct((BF, 1), jnp.float32),
            jax.ShapeDtypeStruct((BF, D), jnp.float32),
        ),
        mesh=mesh,
        compiler_params=pltpu.CompilerParams(use_tc_tiling_on_sc=False),
        scratch_types=[
            pltpu.VMEM((PW,), jnp.int32),
            pltpu.VMEM((PW,), jnp.int32),
            pltpu.VMEM((CHUNK, D), jnp.float32),
            pltpu.VMEM((CHUNK, D), jnp.float32),
            pltpu.VMEM((CHUNK, 1), jnp.float32),
            pltpu.VMEM((CHUNK, 1), jnp.float32),
            pltpu.SemaphoreType.DMA,
            pltpu.SemaphoreType.DMA,
            pltpu.SemaphoreType.DMA,
            pltpu.SemaphoreType.DMA,
        ],
    )(x_flat, emb1_w, emb2_w)


BB = 512


def _mlp_body(w_ref, v_ref, w1w_ref, w1v_ref, b1_ref, w2_ref, b2_ref,
              w3_ref, b3_ref, w4_ref, b4_ref, out_ref):
    f32 = jnp.float32
    h = jnp.dot(w_ref[...], w1w_ref[...], preferred_element_type=f32)
    h = h + jnp.dot(v_ref[...], w1v_ref[...], preferred_element_type=f32)
    h = jax.nn.relu(h + b1_ref[...])
    h = jax.nn.relu(jnp.dot(h, w2_ref[...], preferred_element_type=f32)
                    + b2_ref[...])
    h = jax.nn.relu(jnp.dot(h, w3_ref[...], preferred_element_type=f32)
                    + b3_ref[...])
    o = jnp.dot(h, w4_ref[...], preferred_element_type=f32) + b4_ref[...]
    out_ref[...] = jax.nn.sigmoid(o)


def _mlp(w2d, v2d, W1w, W1v, b1, W2, b2, W3, b3, W4, b4):
    full = lambda s: pl.BlockSpec(s, lambda i: (0, 0))
    return pl.pallas_call(
        _mlp_body,
        grid=(B // BB,),
        in_specs=[
            pl.BlockSpec((BB, F), lambda i: (i, 0)),
            pl.BlockSpec((BB, F * D), lambda i: (i, 0)),
            full(W1w.shape), full(W1v.shape), full(b1.shape),
            full(W2.shape), full(b2.shape),
            full(W3.shape), full(b3.shape),
            full(W4.shape), full(b4.shape),
        ],
        out_specs=pl.BlockSpec((BB, 1), lambda i: (i, 0)),
        out_shape=jax.ShapeDtypeStruct((B, 1), jnp.float32),
    )(w2d, v2d, W1w, W1v, b1, W2, b2, W3, b3, W4, b4)


def kernel(x, emb1_w, emb2_w, W1, b1, W2, b2, W3, b3, W4, b4):
    x_flat = x.reshape(-1)
    w_flat, v_flat = _emb_gather(x_flat, emb1_w, emb2_w)
    w2d = w_flat.reshape(B, F)
    v2d = v_flat.reshape(B, F * D)
    return _mlp(w2d, v2d, W1[:F], W1[F:], b1.reshape(1, -1),
                W2, b2.reshape(1, -1), W3, b3.reshape(1, -1),
                W4, b4.reshape(1, -1))


# speedup vs baseline: 3.4051x; 3.4051x over previous
import jax
import jax.numpy as jnp
from jax import lax
from jax.experimental import pallas as pl
from jax.experimental.pallas import tpu as pltpu
from jax.experimental.pallas import tpu_sc as plsc

B = 4096
F = 26
D = 64
ROWS_PER_TABLE = 100000
R = F * ROWS_PER_TABLE
BF = B * F

NC, NS = 2, 16
NW = NC * NS
PW = BF // NW
CHUNK = 128
NCHUNK = PW // CHUNK


def _emb_gather_body(idx3, q3, emb1f, emb2p, w_hbm, v_hbm,
                     idxv, qv, wv, dstb, s2a, s2b, s1):
    wid = lax.axis_index("s") * NC + lax.axis_index("c")
    base = pl.multiple_of(wid * PW, PW)
    pltpu.sync_copy(idx3.at[wid], idxv)
    pltpu.sync_copy(q3.at[wid], qv)

    def fire(c, slot, sem):
        pltpu.async_copy(emb2p.at[qv.at[c]], dstb.at[slot], sem)
        pltpu.async_copy(emb1f.at[idxv.at[c]],
                         wv.at[pl.ds(c * CHUNK, CHUNK)], s1)

    def drain(c, slot, sem):
        v_dst = v_hbm.at[pl.ds(base + c * CHUNK, CHUNK)]
        pltpu.make_async_copy(v_dst, dstb.at[slot], sem).wait()
        pltpu.sync_copy(dstb.at[slot], v_dst)

    def body(p, carry):
        fire(2 * p, 0, s2a)

        @pl.when(p > 0)
        def _():
            drain(2 * p - 1, 1, s2b)

        fire(2 * p + 1, 1, s2b)
        drain(2 * p, 0, s2a)
        return carry

    lax.fori_loop(0, NCHUNK // 2, body, 0)
    drain(NCHUNK - 1, 1, s2b)

    pltpu.make_async_copy(
        emb1f.at[pl.ds(0, PW)], wv.at[pl.ds(0, PW)], s1).wait()
    pltpu.sync_copy(wv, w_hbm.at[pl.ds(base, PW)])


def _emb_gather(idx3, q3, emb1_flat, emb2p):
    mesh = plsc.VectorSubcoreMesh(
        core_axis_name="c", subcore_axis_name="s", num_cores=NC,
        num_subcores=NS)
    return pl.kernel(
        _emb_gather_body,
        out_type=(
            jax.ShapeDtypeStruct((BF,), jnp.float32),
            jax.ShapeDtypeStruct((BF, 2 * D), jnp.float32),
        ),
        mesh=mesh,
        scratch_types=[
            pltpu.VMEM((NCHUNK, CHUNK), jnp.int32),
            pltpu.VMEM((NCHUNK, CHUNK), jnp.int32),
            pltpu.VMEM((PW,), jnp.float32),
            pltpu.VMEM((2, CHUNK, 2 * D), jnp.float32),
            pltpu.SemaphoreType.DMA,
            pltpu.SemaphoreType.DMA,
            pltpu.SemaphoreType.DMA,
        ],
    )(idx3, q3, emb1_flat, emb2p)


BB = 512


def _mlp_body(wT_ref, v_ref, p_ref, w1w_ref, w1v_ref, b1_ref, w2_ref,
              b2_ref, w3_ref, b3_ref, w4_ref, b4_ref, out_ref):
    f32 = jnp.float32
    dn = (((0,), (0,)), ((), ()))
    h = lax.dot_general(wT_ref[...], w1w_ref[...], dn,
                        preferred_element_type=f32)
    for f in range(F):
        ve = v_ref[f, :, :D]
        vo = v_ref[f, :, D:]
        vsel = ve + p_ref[f] * (vo - ve)
        h = h + jnp.dot(vsel, w1v_ref[f], preferred_element_type=f32)
    h = jax.nn.relu(h + b1_ref[...])
    h = jax.nn.relu(jnp.dot(h, w2_ref[...], preferred_element_type=f32)
                    + b2_ref[...])
    h = jax.nn.relu(jnp.dot(h, w3_ref[...], preferred_element_type=f32)
                    + b3_ref[...])
    o = jnp.dot(h, w4_ref[...], preferred_element_type=f32) + b4_ref[...]
    out_ref[...] = jax.nn.sigmoid(o)


def _mlp(wT, v4, parity, W1w, W1v, b1, W2, b2, W3, b3, W4, b4):
    full = lambda s: pl.BlockSpec(s, lambda i: tuple(0 for _ in s))
    return pl.pallas_call(
        _mlp_body,
        grid=(B // BB,),
        in_specs=[
            pl.BlockSpec((F, BB), lambda i: (0, i)),
            pl.BlockSpec((F, BB, 2 * D), lambda i: (0, i, 0)),
            pl.BlockSpec((F, BB, 1), lambda i: (0, i, 0)),
            full(W1w.shape), full(W1v.shape), full(b1.shape),
            full(W2.shape), full(b2.shape),
            full(W3.shape), full(b3.shape),
            full(W4.shape), full(b4.shape),
        ],
        out_specs=pl.BlockSpec((BB, 1), lambda i: (i, 0)),
        out_shape=jax.ShapeDtypeStruct((B, 1), jnp.float32),
    )(wT, v4, parity, W1w, W1v, b1, W2, b2, W3, b3, W4, b4)


def kernel(x, emb1_w, emb2_w, W1, b1, W2, b2, W3, b3, W4, b4):
    offsets = jnp.arange(F, dtype=jnp.int32) * ROWS_PER_TABLE
    idx = x.T + offsets[:, None]
    idx3 = idx.reshape(NW, NCHUNK, CHUNK)
    q3 = (idx >> 1).reshape(NW, NCHUNK, CHUNK)
    parity = (idx & 1).astype(jnp.float32).reshape(F, B, 1)
    emb1_flat = emb1_w.reshape(R)
    emb2p = emb2_w.reshape(R // 2, 2 * D)
    w_flat, v_flat = _emb_gather(idx3, q3, emb1_flat, emb2p)
    wT = w_flat.reshape(F, B)
    v4 = v_flat.reshape(F, B, 2 * D)
    return _mlp(wT, v4, parity, W1[:F], W1[F:].reshape(F, D, 128),
                b1.reshape(1, -1), W2, b2.reshape(1, -1),
                W3, b3.reshape(1, -1), W4, b4.reshape(1, -1))


# speedup vs baseline: 5.2390x; 1.5386x over previous
import jax
import jax.numpy as jnp
from jax import lax
from jax.experimental import pallas as pl
from jax.experimental.pallas import tpu as pltpu
from jax.experimental.pallas import tpu_sc as plsc

B = 4096
F = 26
D = 64
ROWS_PER_TABLE = 100000
R = F * ROWS_PER_TABLE
BF = B * F

NC, NS = 2, 16
NW = NC * NS
PW = BF // NW
CHUNK = 128
NCHUNK = PW // CHUNK


def _emb_gather_body(idx3, q3, emb1f, emb2p, w_hbm, v_hbm,
                     idxv, qv, wv, dstb, s2a, s2b, s1):
    wid = lax.axis_index("s") * NC + lax.axis_index("c")
    base = pl.multiple_of(wid * PW, PW)
    pltpu.sync_copy(idx3.at[wid], idxv)
    pltpu.sync_copy(q3.at[wid], qv)

    def fire(c, slot, sem):
        pltpu.async_copy(emb2p.at[qv.at[c]], dstb.at[slot], sem)
        pltpu.async_copy(emb1f.at[idxv.at[c]],
                         wv.at[pl.ds(c * CHUNK, CHUNK)], s1)

    def drain(c, slot, sem):
        v_dst = v_hbm.at[pl.ds(base + c * CHUNK, CHUNK)]
        pltpu.make_async_copy(v_dst, dstb.at[slot], sem).wait()
        pltpu.sync_copy(dstb.at[slot], v_dst)

    def body(p, carry):
        fire(2 * p, 0, s2a)

        @pl.when(p > 0)
        def _():
            drain(2 * p - 1, 1, s2b)

        fire(2 * p + 1, 1, s2b)
        drain(2 * p, 0, s2a)
        return carry

    lax.fori_loop(0, NCHUNK // 2, body, 0)
    drain(NCHUNK - 1, 1, s2b)

    pltpu.make_async_copy(
        emb1f.at[pl.ds(0, PW)], wv.at[pl.ds(0, PW)], s1).wait()
    pltpu.sync_copy(wv, w_hbm.at[pl.ds(base, PW)])


def _emb_gather(idx3, q3, emb1_flat, emb2p):
    mesh = plsc.VectorSubcoreMesh(
        core_axis_name="c", subcore_axis_name="s", num_cores=NC,
        num_subcores=NS)
    return pl.kernel(
        _emb_gather_body,
        out_type=(
            jax.ShapeDtypeStruct((BF,), jnp.float32),
            jax.ShapeDtypeStruct((BF, 2 * D), jnp.float32),
        ),
        mesh=mesh,
        scratch_types=[
            pltpu.VMEM((NCHUNK, CHUNK), jnp.int32),
            pltpu.VMEM((NCHUNK, CHUNK), jnp.int32),
            pltpu.VMEM((PW,), jnp.float32),
            pltpu.VMEM((2, CHUNK, 2 * D), jnp.float32),
            pltpu.SemaphoreType.DMA,
            pltpu.SemaphoreType.DMA,
            pltpu.SemaphoreType.DMA,
        ],
    )(idx3, q3, emb1_flat, emb2p)


K = 1304576
BQ = 2048
NBI = K // BQ
LAST_IN_BLOCK = (R - 1) // BQ


def _detile_body(a_ref, b_ref, out_ref):
    out_ref[:, :D] = a_ref[...].T
    out_ref[:, D:] = b_ref[...].T


def _detile(emb2T):
    return pl.pallas_call(
        _detile_body,
        grid=(NBI,),
        in_specs=[
            pl.BlockSpec((D, BQ), lambda i: (0, i)),
            pl.BlockSpec((D, BQ), lambda i: (0, jnp.minimum(i + NBI,
                                                            LAST_IN_BLOCK))),
        ],
        out_specs=pl.BlockSpec((BQ, 2 * D), lambda i: (i, 0)),
        out_shape=jax.ShapeDtypeStruct((K, 2 * D), jnp.float32),
    )(emb2T, emb2T)


BB = 512


def _mlp_body(wT_ref, v_ref, p_ref, w1w_ref, w1v_ref, b1_ref, w2_ref,
              b2_ref, w3_ref, b3_ref, w4_ref, b4_ref, out_ref):
    f32 = jnp.float32
    dn = (((0,), (0,)), ((), ()))
    h = lax.dot_general(wT_ref[...], w1w_ref[...], dn,
                        preferred_element_type=f32)
    for f in range(F):
        ve = v_ref[f, :, :D]
        vo = v_ref[f, :, D:]
        vsel = ve + p_ref[f] * (vo - ve)
        h = h + jnp.dot(vsel, w1v_ref[f], preferred_element_type=f32)
    h = jax.nn.relu(h + b1_ref[...])
    h = jax.nn.relu(jnp.dot(h, w2_ref[...], preferred_element_type=f32)
                    + b2_ref[...])
    h = jax.nn.relu(jnp.dot(h, w3_ref[...], preferred_element_type=f32)
                    + b3_ref[...])
    o = jnp.dot(h, w4_ref[...], preferred_element_type=f32) + b4_ref[...]
    out_ref[...] = jax.nn.sigmoid(o)


def _mlp(wT, v4, parity, W1w, W1v, b1, W2, b2, W3, b3, W4, b4):
    full = lambda s: pl.BlockSpec(s, lambda i: tuple(0 for _ in s))
    return pl.pallas_call(
        _mlp_body,
        grid=(B // BB,),
        in_specs=[
            pl.BlockSpec((F, BB), lambda i: (0, i)),
            pl.BlockSpec((F, BB, 2 * D), lambda i: (0, i, 0)),
            pl.BlockSpec((F, BB, 1), lambda i: (0, i, 0)),
            full(W1w.shape), full(W1v.shape), full(b1.shape),
            full(W2.shape), full(b2.shape),
            full(W3.shape), full(b3.shape),
            full(W4.shape), full(b4.shape),
        ],
        out_specs=pl.BlockSpec((BB, 1), lambda i: (i, 0)),
        out_shape=jax.ShapeDtypeStruct((B, 1), jnp.float32),
    )(wT, v4, parity, W1w, W1v, b1, W2, b2, W3, b3, W4, b4)


def kernel(x, emb1_w, emb2_w, W1, b1, W2, b2, W3, b3, W4, b4):
    offsets = jnp.arange(F, dtype=jnp.int32) * ROWS_PER_TABLE
    idx = x.T + offsets[:, None]
    idx3 = idx.reshape(NW, NCHUNK, CHUNK)
    hi = idx >= K
    q3 = jnp.where(hi, idx - K, idx).reshape(NW, NCHUNK, CHUNK)
    parity = hi.astype(jnp.float32).reshape(F, B, 1)
    emb1_flat = emb1_w.reshape(R)
    emb2p = _detile(emb2_w.T)
    w_flat, v_flat = _emb_gather(idx3, q3, emb1_flat, emb2p)
    wT = w_flat.reshape(F, B)
    v4 = v_flat.reshape(F, B, 2 * D)
    return _mlp(wT, v4, parity, W1[:F], W1[F:].reshape(F, D, 128),
                b1.reshape(1, -1), W2, b2.reshape(1, -1),
                W3, b3.reshape(1, -1), W4, b4.reshape(1, -1))


# speedup vs baseline: 5.5326x; 1.0560x over previous
import jax
import jax.numpy as jnp
from jax import lax
from jax.experimental import pallas as pl
from jax.experimental.pallas import tpu as pltpu
from jax.experimental.pallas import tpu_sc as plsc

B = 4096
F = 26
D = 64
ROWS_PER_TABLE = 100000
R = F * ROWS_PER_TABLE
BF = B * F

NC, NS = 2, 16
NW = NC * NS
PW = BF // NW
CHUNK = 128
NCHUNK = PW // CHUNK


def _emb_gather_body(idx3, q3, emb1f, emb2p, w_hbm, v_hbm,
                     idxv, qv, wv, dstb, s2a, s2b, s1):
    wid = lax.axis_index("s") * NC + lax.axis_index("c")
    base = pl.multiple_of(wid * PW, PW)
    pltpu.sync_copy(idx3.at[wid], idxv)
    pltpu.sync_copy(q3.at[wid], qv)

    def fire(c, slot, sem):
        pltpu.async_copy(emb2p.at[qv.at[c]], dstb.at[slot], sem)
        pltpu.async_copy(emb1f.at[idxv.at[c]],
                         wv.at[pl.ds(c * CHUNK, CHUNK)], s1)

    def drain(c, slot, sem):
        v_dst = v_hbm.at[pl.ds(base + c * CHUNK, CHUNK)]
        pltpu.make_async_copy(v_dst, dstb.at[slot], sem).wait()
        pltpu.sync_copy(dstb.at[slot], v_dst)

    def body(p, carry):
        fire(2 * p, 0, s2a)

        @pl.when(p > 0)
        def _():
            drain(2 * p - 1, 1, s2b)

        fire(2 * p + 1, 1, s2b)
        drain(2 * p, 0, s2a)
        return carry

    lax.fori_loop(0, NCHUNK // 2, body, 0)
    drain(NCHUNK - 1, 1, s2b)

    pltpu.make_async_copy(
        emb1f.at[pl.ds(0, PW)], wv.at[pl.ds(0, PW)], s1).wait()
    pltpu.sync_copy(wv, w_hbm.at[pl.ds(base, PW)])


def _emb_gather(idx3, q3, emb1_flat, emb2p):
    mesh = plsc.VectorSubcoreMesh(
        core_axis_name="c", subcore_axis_name="s", num_cores=NC,
        num_subcores=NS)
    return pl.kernel(
        _emb_gather_body,
        out_type=(
            jax.ShapeDtypeStruct((BF,), jnp.float32),
            jax.ShapeDtypeStruct((BF, 2 * D), jnp.int32),
        ),
        mesh=mesh,
        scratch_types=[
            pltpu.VMEM((NCHUNK, CHUNK), jnp.int32),
            pltpu.VMEM((NCHUNK, CHUNK), jnp.int32),
            pltpu.VMEM((PW,), jnp.float32),
            pltpu.VMEM((2, CHUNK, 2 * D), jnp.int32),
            pltpu.SemaphoreType.DMA,
            pltpu.SemaphoreType.DMA,
            pltpu.SemaphoreType.DMA,
        ],
    )(idx3, q3, emb1_flat, emb2p)


K2 = 652288
BQ = 1024
NBI = K2 // BQ
LAST_IN_BLOCK = (R - 1) // BQ


def _pack_hi(t):
    u = lax.bitcast_convert_type(t, jnp.uint32)
    return (u + jnp.uint32(0x8000)) & jnp.uint32(0xFFFF0000)


def _detile_body(a_ref, b_ref, c_ref, d_ref, out_ref):
    left = _pack_hi(a_ref[...].T) | (_pack_hi(b_ref[...].T) >> 16)
    right = _pack_hi(c_ref[...].T) | (_pack_hi(d_ref[...].T) >> 16)
    out_ref[:, :D] = lax.bitcast_convert_type(left, jnp.int32)
    out_ref[:, D:] = lax.bitcast_convert_type(right, jnp.int32)


def _detile(emb2T):
    return pl.pallas_call(
        _detile_body,
        grid=(NBI,),
        in_specs=[
            pl.BlockSpec((D, BQ), lambda i: (0, i)),
            pl.BlockSpec((D, BQ), lambda i: (0, i + NBI)),
            pl.BlockSpec((D, BQ), lambda i: (0, i + 2 * NBI)),
            pl.BlockSpec((D, BQ), lambda i: (0, jnp.minimum(i + 3 * NBI,
                                                            LAST_IN_BLOCK))),
        ],
        out_specs=pl.BlockSpec((BQ, 2 * D), lambda i: (i, 0)),
        out_shape=jax.ShapeDtypeStruct((K2, 2 * D), jnp.int32),
    )(emb2T, emb2T, emb2T, emb2T)


BB = 512


def _mlp_body(wT_ref, v_ref, p0_ref, p1_ref, w1w_ref, w1v_ref, b1_ref,
              w2_ref, b2_ref, w3_ref, b3_ref, w4_ref, b4_ref, out_ref):
    f32 = jnp.float32
    dn = (((0,), (0,)), ((), ()))
    h = lax.dot_general(wT_ref[...], w1w_ref[...], dn,
                        preferred_element_type=f32)
    for f in range(F):
        xu = lax.bitcast_convert_type(v_ref[f], jnp.uint32)
        hi = lax.bitcast_convert_type(xu & jnp.uint32(0xFFFF0000), f32)
        lo = lax.bitcast_convert_type(xu << 16, f32)
        p0 = p0_ref[f]
        vlow = hi[:, :D] + p0 * (lo[:, :D] - hi[:, :D])
        vhigh = hi[:, D:] + p0 * (lo[:, D:] - hi[:, D:])
        vsel = vlow + p1_ref[f] * (vhigh - vlow)
        h = h + jnp.dot(vsel, w1v_ref[f], preferred_element_type=f32)
    h = jax.nn.relu(h + b1_ref[...])
    h = jax.nn.relu(jnp.dot(h, w2_ref[...], preferred_element_type=f32)
                    + b2_ref[...])
    h = jax.nn.relu(jnp.dot(h, w3_ref[...], preferred_element_type=f32)
                    + b3_ref[...])
    o = jnp.dot(h, w4_ref[...], preferred_element_type=f32) + b4_ref[...]
    out_ref[...] = jax.nn.sigmoid(o)


def _mlp(wT, v4, p0, p1, W1w, W1v, b1, W2, b2, W3, b3, W4, b4):
    full = lambda s: pl.BlockSpec(s, lambda i: tuple(0 for _ in s))
    return pl.pallas_call(
        _mlp_body,
        grid=(B // BB,),
        in_specs=[
            pl.BlockSpec((F, BB), lambda i: (0, i)),
            pl.BlockSpec((F, BB, 2 * D), lambda i: (0, i, 0)),
            pl.BlockSpec((F, BB, 1), lambda i: (0, i, 0)),
            pl.BlockSpec((F, BB, 1), lambda i: (0, i, 0)),
            full(W1w.shape), full(W1v.shape), full(b1.shape),
            full(W2.shape), full(b2.shape),
            full(W3.shape), full(b3.shape),
            full(W4.shape), full(b4.shape),
        ],
        out_specs=pl.BlockSpec((BB, 1), lambda i: (i, 0)),
        out_shape=jax.ShapeDtypeStruct((B, 1), jnp.float32),
    )(wT, v4, p0, p1, W1w, W1v, b1, W2, b2, W3, b3, W4, b4)


def kernel(x, emb1_w, emb2_w, W1, b1, W2, b2, W3, b3, W4, b4):
    offsets = jnp.arange(F, dtype=jnp.int32) * ROWS_PER_TABLE
    idx = x.T + offsets[:, None]
    idx3 = idx.reshape(NW, NCHUNK, CHUNK)
    sel = idx // K2
    q3 = (idx - sel * K2).reshape(NW, NCHUNK, CHUNK)
    p0 = (sel & 1).astype(jnp.float32).reshape(F, B, 1)
    p1 = (sel >> 1).astype(jnp.float32).reshape(F, B, 1)
    emb1_flat = emb1_w.reshape(R)
    emb2q = _detile(emb2_w.T)
    w_flat, v_flat = _emb_gather(idx3, q3, emb1_flat, emb2q)
    wT = w_flat.reshape(F, B)
    v4 = v_flat.reshape(F, B, 2 * D)
    return _mlp(wT, v4, p0, p1, W1[:F], W1[F:].reshape(F, D, 128),
                b1.reshape(1, -1), W2, b2.reshape(1, -1),
                W3, b3.reshape(1, -1), W4, b4.reshape(1, -1))
